# baseline (speedup 1.0000x reference)
import functools

import jax
import jax.numpy as jnp
from jax import lax
from jax.experimental import pallas as pl
from jax.experimental.pallas import tpu as pltpu
from jax.experimental.pallas import tpu_sc as plsc

N = 16384
D = 64
_NNZ_C = 268435
_NNZ_U = 65536

_NUM_SUBCORES = 16
_LANES = 16
_DH = D // 2

_WIN = 128
_WPG = 4
_GROUP = _WIN * _WPG
_ROWS_PER_TILE = N // _NUM_SUBCORES

_CT_WINS = _NNZ_C // _WIN
_CT_FULL = _CT_WINS * _WIN
_CT_TAIL = _NNZ_C - _CT_FULL
_A0X_LEN = _NNZ_U + _WIN
_A0X_WINS = _A0X_LEN // _WIN
_A1_WINS = _NNZ_U // _WIN

_mesh = plsc.VectorSubcoreMesh(core_axis_name="c", subcore_axis_name="s")


@functools.partial(
    pl.kernel,
    out_type=(
        jax.ShapeDtypeStruct((N, D), jnp.float32),
        jax.ShapeDtypeStruct((N, D), jnp.float32),
    ),
    mesh=_mesh,
    scratch_types=(
        pltpu.VMEM((3, _WPG, _WIN), jnp.int32),
        pltpu.VMEM((3, _WPG, _WIN), jnp.int32),
        pltpu.VMEM((3, _GROUP), jnp.float32),
        pltpu.VMEM((_GROUP, _DH), jnp.float32),
        pltpu.VMEM((_GROUP, _DH), jnp.float32),
        pltpu.VMEM((_GROUP, _DH), jnp.float32),
        pltpu.VMEM_SHARED((N, _DH), jnp.float32),
        pltpu.SemaphoreType.DMA,
        pltpu.SemaphoreType.DMA,
        pltpu.SemaphoreType.DMA,
        pltpu.SemaphoreType.DMA,
        pltpu.SemaphoreType.DMA,
        pltpu.SemaphoreType.DMA,
        pltpu.SemaphoreType.DMA,
        pltpu.SemaphoreType.DMA,
        pltpu.SemaphoreType.DMA,
    ),
    compiler_params=pltpu.CompilerParams(use_tc_tiling_on_sc=False),
)
def _sc_spmm(basis_lo, basis_hi, ct_idx, ct_vals,
             a0x_rows, a0x_cols, a0x_vals, a1_idx, a1_vals,
             out_r, out_i,
             rows_b, cols_b, vals_b, gath0, gath1, gath2, acc,
             gsem0, gsem1, gsem2, ssem0, ssem1, ssem2,
             isem0, isem1, isem2):
    cid = lax.axis_index("c")
    sid = lax.axis_index("s")
    rsl = pl.ds(sid * _ROWS_PER_TILE, _ROWS_PER_TILE)
    zero16f = jnp.zeros((_LANES,), jnp.float32)
    gaths = (gath0, gath1, gath2)
    gsems = (gsem0, gsem1, gsem2)
    ssems = (ssem0, ssem1, ssem2)
    isems = (isem0, isem1, isem2)

    def zero_acc_slice():
        def zrow(r, carry):
            for c in range(_DH // _LANES):
                gath0[r, pl.ds(c * _LANES, _LANES)] = zero16f
            return carry

        lax.fori_loop(0, _GROUP, zrow, 0)
        for h in range(_ROWS_PER_TILE // _GROUP):
            pltpu.sync_copy(
                gath0,
                acc.at[pl.ds(sid * _ROWS_PER_TILE + h * _GROUP, _GROUP), :])

    def run_segment(basis, rows_h, cols_h, vals_h, total_wins):
        per = -(-total_wins // _NUM_SUBCORES)
        ngroups = -(-per // _WPG)
        base = sid * per
        my_nw = jnp.clip(total_wins - base, 0, per)
        lim_off = (total_wins - 1) * _WIN

        def drain_gath(sem):
            pltpu.make_async_copy(
                basis.at[pl.ds(0, _GROUP), :], gath0, sem).wait()

        def drain_idx(b):
            for _ in range(3):
                pltpu.make_async_copy(
                    vals_h.at[pl.ds(0, _GROUP)], vals_b.at[b], isems[b]).wait()

        def stage_idx(b, g, may_drain):
            if may_drain:
                @pl.when(g >= 3)
                def _():
                    drain_gath(ssems[b])
            for j in range(_WPG):
                wj = g * _WPG + j
                woff = jnp.minimum((base + wj) * _WIN, lim_off)
                pltpu.async_copy(
                    rows_h.at[pl.ds(woff, _WIN)], rows_b.at[b, j], isems[b])
                pltpu.async_copy(
                    cols_h.at[pl.ds(woff, _WIN)], cols_b.at[b, j], isems[b])
                pltpu.async_copy(
                    vals_h.at[pl.ds(woff, _WIN)],
                    vals_b.at[b, pl.ds(j * _WIN, _WIN)], isems[b])

        def stage_gath(b, g):
            drain_idx(b)
            for j in range(_WPG):
                ok = (g * _WPG + j) < my_nw

                @pl.when(jnp.logical_not(ok))
                def _():
                    for k in range(_WIN // _LANES):
                        vals_b[b, pl.ds(j * _WIN + k * _LANES, _LANES)] = (
                            zero16f)
            for j in range(_WPG):
                pltpu.async_copy(
                    basis.at[cols_b.at[b, j]],
                    gaths[b].at[pl.ds(j * _WIN, _WIN), :], gsems[b])

        def stage_comp(b):
            drain_gath(gsems[b])
            gath = gaths[b]

            def srow16(r0, inner):
                v16 = vals_b[b, pl.ds(r0 * _LANES, _LANES)]
                for l in range(_LANES):
                    r = r0 * _LANES + l
                    vv = v16[lax.full((_LANES,), l, jnp.int32)]
                    for c in range(_DH // _LANES):
                        sl = pl.ds(c * _LANES, _LANES)
                        gath[r, sl] = gath[r, sl] * vv
                return inner

            lax.fori_loop(0, _GROUP // _LANES, srow16, 0)
            for j in range(_WPG):
                pltpu.async_copy(
                    gath.at[pl.ds(j * _WIN, _WIN), :],
                    acc.at[rows_b.at[b, j]], ssems[b], add=True)

        if ngroups >= 1:
            stage_idx(0, 0, may_drain=False)
        if ngroups >= 2:
            stage_idx(1, 1, may_drain=False)
        stage_gath(0, 0)

        def triple(p, carry):
            for o in (0, 1, 2):
                g = 3 * p + o

                @pl.when(g + 1 < ngroups)
                def _():
                    stage_gath((o + 1) % 3, g + 1)

                @pl.when(g < ngroups)
                def _():
                    stage_comp(o)

                @pl.when(g + 2 < ngroups)
                def _():
                    stage_idx((o + 2) % 3, g + 2, may_drain=True)
            return carry

        lax.fori_loop(0, -(-ngroups // 3), triple, 0)
        for k in range(max(0, ngroups - 3), ngroups):
            drain_gath(ssems[k % 3])

    def writeback(out, colofs):
        for h in range(_ROWS_PER_TILE // _GROUP):
            row0 = sid * _ROWS_PER_TILE + h * _GROUP
            buf = gaths[h % 2]
            pltpu.sync_copy(acc.at[pl.ds(row0, _GROUP), :], buf)
            pltpu.sync_copy(buf, out.at[pl.ds(row0, _GROUP),
                                        pl.ds(colofs, _DH)])

    def full_flow(basis, colofs):
        zero_acc_slice()
        plsc.subcore_barrier()
        run_segment(basis, ct_idx.at[1], ct_idx.at[0], ct_vals, _CT_WINS)
        run_segment(basis, a0x_rows, a0x_cols, a0x_vals, _A0X_WINS)
        plsc.subcore_barrier()
        writeback(out_r, colofs)
        zero_acc_slice()
        plsc.subcore_barrier()
        run_segment(basis, a1_idx.at[0], a1_idx.at[1], a1_vals, _A1_WINS)
        plsc.subcore_barrier()
        writeback(out_i, colofs)

    @pl.when(cid == 0)
    def _():
        full_flow(basis_lo, 0)

    @pl.when(cid == 1)
    def _():
        full_flow(basis_hi, _DH)


def kernel(atomic_basis, c_tilde_indices, c_tilde_values,
           a_update0_indices, a_update0_values,
           a_update1_indices, a_update1_values):
    pad_i = jnp.zeros((_WIN - _CT_TAIL,), jnp.int32)
    a0x_rows = jnp.concatenate(
        [a_update0_indices[0], c_tilde_indices[1, _CT_FULL:], pad_i])
    a0x_cols = jnp.concatenate(
        [a_update0_indices[1], c_tilde_indices[0, _CT_FULL:], pad_i])
    a0x_vals = jnp.concatenate(
        [a_update0_values, c_tilde_values[_CT_FULL:],
         jnp.zeros((_WIN - _CT_TAIL,), jnp.float32)])

    basis_lo = atomic_basis[:, :_DH]
    basis_hi = atomic_basis[:, _DH:]

    real_out, imag_out = _sc_spmm(
        basis_lo, basis_hi, c_tilde_indices, c_tilde_values,
        a0x_rows, a0x_cols, a0x_vals, a_update1_indices, a_update1_values)
    return (real_out, imag_out)


# speedup vs baseline: 1.0811x; 1.0811x over previous
import functools

import jax
import jax.numpy as jnp
from jax import lax
from jax.experimental import pallas as pl
from jax.experimental.pallas import tpu as pltpu
from jax.experimental.pallas import tpu_sc as plsc

N = 16384
D = 64
_NNZ_C = 268435
_NNZ_U = 65536

_NUM_SUBCORES = 16
_LANES = 16
_DH = D // 2

_WIN = 128
_WPG = 4
_GROUP = _WIN * _WPG
_ROWS_PER_TILE = N // _NUM_SUBCORES

_CT_WINS = _NNZ_C // _WIN
_CT_FULL = _CT_WINS * _WIN
_CT_TAIL = _NNZ_C - _CT_FULL
_ALIGN = _NUM_SUBCORES * _WPG


def _wpad(nwin):
    return -(-nwin // _ALIGN) * _ALIGN


_CT_P = _wpad(_CT_WINS)
_A0X_P = _wpad(-(-(_NNZ_U + _CT_TAIL) // _WIN))
_A1_P = _wpad(_NNZ_U // _WIN)

_mesh = plsc.VectorSubcoreMesh(core_axis_name="c", subcore_axis_name="s")


@functools.partial(
    pl.kernel,
    out_type=(
        jax.ShapeDtypeStruct((N, D), jnp.float32),
        jax.ShapeDtypeStruct((N, D), jnp.float32),
    ),
    mesh=_mesh,
    scratch_types=(
        pltpu.VMEM((3, _WPG, _WIN), jnp.int32),
        pltpu.VMEM((3, _WPG, _WIN), jnp.int32),
        pltpu.VMEM((3, _WPG, _WIN), jnp.float32),
        pltpu.VMEM((_GROUP, _DH), jnp.float32),
        pltpu.VMEM((_GROUP, _DH), jnp.float32),
        pltpu.VMEM((_GROUP, _DH), jnp.float32),
        pltpu.VMEM_SHARED((N, _DH), jnp.float32),
        pltpu.SemaphoreType.DMA,
        pltpu.SemaphoreType.DMA,
        pltpu.SemaphoreType.DMA,
        pltpu.SemaphoreType.DMA,
        pltpu.SemaphoreType.DMA,
        pltpu.SemaphoreType.DMA,
        pltpu.SemaphoreType.DMA,
        pltpu.SemaphoreType.DMA,
        pltpu.SemaphoreType.DMA,
    ),
    compiler_params=pltpu.CompilerParams(use_tc_tiling_on_sc=False),
)
def _sc_spmm(basis_lo, basis_hi, ct_r, ct_c, ct_v,
             a0x_r, a0x_c, a0x_v, a1_r, a1_c, a1_v,
             out_r, out_i,
             rows_b, cols_b, vals_b, gath0, gath1, gath2, acc,
             gsem0, gsem1, gsem2, ssem0, ssem1, ssem2,
             isem0, isem1, isem2):
    cid = lax.axis_index("c")
    sid = lax.axis_index("s")
    zero16f = jnp.zeros((_LANES,), jnp.float32)
    gaths = (gath0, gath1, gath2)
    gsems = (gsem0, gsem1, gsem2)
    ssems = (ssem0, ssem1, ssem2)
    isems = (isem0, isem1, isem2)

    def zero_acc_slice():
        def zrow(r, carry):
            for c in range(_DH // _LANES):
                gath0[r, pl.ds(c * _LANES, _LANES)] = zero16f
            return carry

        lax.fori_loop(0, _GROUP, zrow, 0)
        for h in range(_ROWS_PER_TILE // _GROUP):
            pltpu.sync_copy(
                gath0,
                acc.at[pl.ds(sid * _ROWS_PER_TILE + h * _GROUP, _GROUP), :])

    def run_segment(basis, r2, c2, v2, per):
        ngroups = per // _WPG
        base = sid * per

        def drain_gath(sem):
            pltpu.make_async_copy(
                basis.at[pl.ds(0, _GROUP), :], gath0, sem).wait()

        def drain_idx(b):
            for _ in range(3):
                pltpu.make_async_copy(
                    v2.at[pl.ds(0, _WPG), :], vals_b.at[b], isems[b]).wait()

        def stage_idx(b, g, may_drain):
            if may_drain:
                @pl.when(g >= 3)
                def _():
                    drain_gath(ssems[b])
            wsl = pl.ds(base + g * _WPG, _WPG)
            pltpu.async_copy(r2.at[wsl, :], rows_b.at[b], isems[b])
            pltpu.async_copy(c2.at[wsl, :], cols_b.at[b], isems[b])
            pltpu.async_copy(v2.at[wsl, :], vals_b.at[b], isems[b])

        def stage_gath(b):
            drain_idx(b)
            for j in range(_WPG):
                pltpu.async_copy(
                    basis.at[cols_b.at[b, j]],
                    gaths[b].at[pl.ds(j * _WIN, _WIN), :], gsems[b])

        def stage_comp(b):
            drain_gath(gsems[b])
            gath = gaths[b]

            def srow16(r0, inner):
                jw = r0 // (_WIN // _LANES)
                kw = r0 % (_WIN // _LANES)
                v16 = vals_b[b, jw, pl.ds(kw * _LANES, _LANES)]
                for l in range(_LANES):
                    r = r0 * _LANES + l
                    vv = v16[lax.full((_LANES,), l, jnp.int32)]
                    for c in range(_DH // _LANES):
                        sl = pl.ds(c * _LANES, _LANES)
                        gath[r, sl] = gath[r, sl] * vv
                return inner

            lax.fori_loop(0, _GROUP // _LANES, srow16, 0)
            for j in range(_WPG):
                pltpu.async_copy(
                    gath.at[pl.ds(j * _WIN, _WIN), :],
                    acc.at[rows_b.at[b, j]], ssems[b], add=True)

        stage_idx(0, 0, may_drain=False)
        if ngroups >= 2:
            stage_idx(1, 1, may_drain=False)
        stage_gath(0)

        def triple(p, carry):
            for o in (0, 1, 2):
                g = 3 * p + o

                @pl.when(g + 1 < ngroups)
                def _():
                    stage_gath((o + 1) % 3)

                @pl.when(g < ngroups)
                def _():
                    stage_comp(o)

                @pl.when(g + 2 < ngroups)
                def _():
                    stage_idx((o + 2) % 3, g + 2, may_drain=True)
            return carry

        lax.fori_loop(0, -(-ngroups // 3), triple, 0)
        for k in range(max(0, ngroups - 3), ngroups):
            drain_gath(ssems[k % 3])

    def writeback(out, colofs):
        for h in range(_ROWS_PER_TILE // _GROUP):
            row0 = sid * _ROWS_PER_TILE + h * _GROUP
            buf = gaths[h % 2]
            pltpu.sync_copy(acc.at[pl.ds(row0, _GROUP), :], buf)
            pltpu.sync_copy(buf, out.at[pl.ds(row0, _GROUP),
                                        pl.ds(colofs, _DH)])

    def full_flow(basis, colofs):
        zero_acc_slice()
        plsc.subcore_barrier()
        run_segment(basis, ct_r, ct_c, ct_v, _CT_P // _NUM_SUBCORES)
        run_segment(basis, a0x_r, a0x_c, a0x_v, _A0X_P // _NUM_SUBCORES)
        plsc.subcore_barrier()
        writeback(out_r, colofs)
        zero_acc_slice()
        plsc.subcore_barrier()
        run_segment(basis, a1_r, a1_c, a1_v, _A1_P // _NUM_SUBCORES)
        plsc.subcore_barrier()
        writeback(out_i, colofs)

    @pl.when(cid == 0)
    def _():
        full_flow(basis_lo, 0)

    @pl.when(cid == 1)
    def _():
        full_flow(basis_hi, _DH)


def _pad2d_i(x_parts, nwin):
    have = sum(p.shape[0] for p in x_parts)
    pad = nwin * _WIN - have
    filler = (jnp.arange(pad, dtype=jnp.int32) % N).astype(jnp.int32)
    return jnp.concatenate(
        [p.astype(jnp.int32) for p in x_parts] + [filler]).reshape(nwin, _WIN)


def _pad2d_f(x_parts, nwin):
    have = sum(p.shape[0] for p in x_parts)
    pad = nwin * _WIN - have
    return jnp.concatenate(
        list(x_parts) + [jnp.zeros((pad,), jnp.float32)]).reshape(nwin, _WIN)


def kernel(atomic_basis, c_tilde_indices, c_tilde_values,
           a_update0_indices, a_update0_values,
           a_update1_indices, a_update1_values):
    ct_r = _pad2d_i([c_tilde_indices[1, :_CT_FULL]], _CT_P)
    ct_c = _pad2d_i([c_tilde_indices[0, :_CT_FULL]], _CT_P)
    ct_v = _pad2d_f([c_tilde_values[:_CT_FULL]], _CT_P)
    a0x_r = _pad2d_i(
        [a_update0_indices[0], c_tilde_indices[1, _CT_FULL:]], _A0X_P)
    a0x_c = _pad2d_i(
        [a_update0_indices[1], c_tilde_indices[0, _CT_FULL:]], _A0X_P)
    a0x_v = _pad2d_f(
        [a_update0_values, c_tilde_values[_CT_FULL:]], _A0X_P)
    a1_r = _pad2d_i([a_update1_indices[0]], _A1_P)
    a1_c = _pad2d_i([a_update1_indices[1]], _A1_P)
    a1_v = _pad2d_f([a_update1_values], _A1_P)

    basis_lo = atomic_basis[:, :_DH]
    basis_hi = atomic_basis[:, _DH:]

    real_out, imag_out = _sc_spmm(
        basis_lo, basis_hi, ct_r, ct_c, ct_v,
        a0x_r, a0x_c, a0x_v, a1_r, a1_c, a1_v)
    return (real_out, imag_out)
